# Initial kernel scaffold; baseline (speedup 1.0000x reference)
import jax
import jax.numpy as jnp
from jax.experimental import pallas as pl


def kernel(bbox, x0_embed, y0_embed, x1_embed, y1_embed, w_embed, h_embed):
    raise NotImplementedError("write your pallas kernel here")


# speedup vs baseline: 1.0861x; 1.0861x over previous
import functools

import jax
import jax.numpy as jnp
from jax import lax
from jax.experimental import pallas as pl
from jax.experimental.pallas import tpu as pltpu
from jax.experimental.pallas import tpu_sc as plsc

B, L, D = 16, 2048, 768
N = B * L
NUM_CORES = 2
NUM_SUBCORES = 16
NW = NUM_CORES * NUM_SUBCORES
TPW = N // NW
C = 16
NCHUNK = TPW // C
NVEC = D // 16


def _sc_body(x0s, y0s, x1s, y1s,
             x0_t, y0_t, x1_t, y1_t, w_t, h_t,
             out_hbm,
             x0_v, y0_v, x1_v, y1_v,
             b0, b1, b2, b3, b4, b5,
             sem_g, sem_o):
    cid = lax.axis_index("c")
    sid = lax.axis_index("s")
    wid = sid * NUM_CORES + cid
    base = wid * TPW

    pltpu.sync_copy(x0s.at[pl.ds(base, TPW)], x0_v)
    pltpu.sync_copy(y0s.at[pl.ds(base, TPW)], y0_v)
    pltpu.sync_copy(x1s.at[pl.ds(base, TPW)], x1_v)
    pltpu.sync_copy(y1s.at[pl.ds(base, TPW)], y1_v)

    def chunk(i, _):
        off = i * C
        x0 = x0_v[pl.ds(off, C)]
        y0 = y0_v[pl.ds(off, C)]
        x1 = x1_v[pl.ds(off, C)]
        y1 = y1_v[pl.ds(off, C)]
        zero = jnp.zeros((C,), jnp.int32)
        hi = jnp.full((C,), 1023, jnp.int32)
        x0c = jnp.minimum(jnp.maximum(x0, zero), hi)
        y0c = jnp.minimum(jnp.maximum(y0, zero), hi)
        x1c = jnp.minimum(jnp.maximum(x1, zero), hi)
        y1c = jnp.minimum(jnp.maximum(y1, zero), hi)
        wc = jnp.minimum(jnp.maximum(x1c - x0c, zero), hi)
        hc = jnp.minimum(jnp.maximum(y1c - y0c, zero), hi)

        cp0 = pltpu.async_copy(x0_t.at[x0c], b0, sem_g)
        cp1 = pltpu.async_copy(y0_t.at[y0c], b1, sem_g)
        cp2 = pltpu.async_copy(x1_t.at[x1c], b2, sem_g)
        cp3 = pltpu.async_copy(y1_t.at[y1c], b3, sem_g)
        cp4 = pltpu.async_copy(w_t.at[wc], b4, sem_g)
        cp5 = pltpu.async_copy(h_t.at[hc], b5, sem_g)
        cp0.wait(); cp1.wait(); cp2.wait(); cp3.wait(); cp4.wait(); cp5.wait()

        def vstep(k, _):
            j = k // NVEC
            v = k % NVEC
            sl = pl.ds(v * 16, 16)
            acc = (b0[j, sl] + b1[j, sl] + b2[j, sl]
                   + b3[j, sl] + b4[j, sl] + b5[j, sl])
            b0[j, sl] = acc
            return 0

        lax.fori_loop(0, C * NVEC, vstep, 0)
        pltpu.sync_copy(b0, out_hbm.at[pl.ds(base + off, C)])
        return 0

    lax.fori_loop(0, NCHUNK, chunk, 0)


@jax.jit
def _run(x0s, y0s, x1s, y1s, x0_t, y0_t, x1_t, y1_t, w_t, h_t):
    mesh = plsc.VectorSubcoreMesh(
        core_axis_name="c", subcore_axis_name="s",
        num_cores=NUM_CORES, num_subcores=NUM_SUBCORES)
    f = pl.kernel(
        _sc_body,
        out_type=jax.ShapeDtypeStruct((N, D), jnp.float32),
        mesh=mesh,
        scratch_types=[
            pltpu.VMEM((TPW,), jnp.int32),
            pltpu.VMEM((TPW,), jnp.int32),
            pltpu.VMEM((TPW,), jnp.int32),
            pltpu.VMEM((TPW,), jnp.int32),
            pltpu.VMEM((C, D), jnp.float32),
            pltpu.VMEM((C, D), jnp.float32),
            pltpu.VMEM((C, D), jnp.float32),
            pltpu.VMEM((C, D), jnp.float32),
            pltpu.VMEM((C, D), jnp.float32),
            pltpu.VMEM((C, D), jnp.float32),
            pltpu.SemaphoreType.DMA,
            pltpu.SemaphoreType.DMA,
        ],
    )
    return f(x0s, y0s, x1s, y1s, x0_t, y0_t, x1_t, y1_t, w_t, h_t)


def kernel(bbox, x0_embed, y0_embed, x1_embed, y1_embed, w_embed, h_embed):
    flat = bbox.reshape(N, 4)
    x0s = flat[:, 0]
    y0s = flat[:, 1]
    x1s = flat[:, 2]
    y1s = flat[:, 3]
    out = _run(x0s, y0s, x1s, y1s,
               x0_embed, y0_embed, x1_embed, y1_embed, w_embed, h_embed)
    return out.reshape(B, L, D)


# speedup vs baseline: 1.0973x; 1.0103x over previous
import jax
import jax.numpy as jnp
from jax import lax
from jax.experimental import pallas as pl
from jax.experimental.pallas import tpu as pltpu
from jax.experimental.pallas import tpu_sc as plsc

B, L, D = 16, 2048, 768
N = B * L
NUM_CORES = 2
NUM_SUBCORES = 16
HD = D // NUM_CORES
TPW = N // NUM_SUBCORES
C = 16
NCHUNK = TPW // C
NVEC = HD // 16


def _sc_body(x0s, y0s, x1s, y1s,
             x0_t, y0_t, x1_t, y1_t, w_t, h_t,
             out_hbm,
             x0_v, y0_v, x1_v, y1_v,
             bufs, sem_g, sem_o):
    hid = lax.axis_index("c")
    tid = lax.axis_index("s")
    base = tid * TPW
    hoff = hid * HD
    tables = (x0_t, y0_t, x1_t, y1_t, w_t, h_t)

    pltpu.sync_copy(x0s.at[pl.ds(base, TPW)], x0_v)
    pltpu.sync_copy(y0s.at[pl.ds(base, TPW)], y0_v)
    pltpu.sync_copy(x1s.at[pl.ds(base, TPW)], x1_v)
    pltpu.sync_copy(y1s.at[pl.ds(base, TPW)], y1_v)

    def indices(i):
        off = i * C
        x0 = x0_v[pl.ds(off, C)]
        y0 = y0_v[pl.ds(off, C)]
        x1 = x1_v[pl.ds(off, C)]
        y1 = y1_v[pl.ds(off, C)]
        zero = jnp.zeros((C,), jnp.int32)
        hi = jnp.full((C,), 1023, jnp.int32)
        x0c = jnp.minimum(jnp.maximum(x0, zero), hi)
        y0c = jnp.minimum(jnp.maximum(y0, zero), hi)
        x1c = jnp.minimum(jnp.maximum(x1, zero), hi)
        y1c = jnp.minimum(jnp.maximum(y1, zero), hi)
        wc = jnp.minimum(jnp.maximum(x1c - x0c, zero), hi)
        hc = jnp.minimum(jnp.maximum(y1c - y0c, zero), hi)
        return (x0c, y0c, x1c, y1c, wc, hc)

    def fire(i, slot):
        idx = indices(i)
        for t in range(6):
            pltpu.async_copy(tables[t].at[idx[t], pl.ds(hoff, HD)],
                             bufs.at[slot, t], sem_g)

    def wait_gathers(i, slot):
        idx = indices(i)
        for t in range(6):
            pltpu.make_async_copy(tables[t].at[idx[t], pl.ds(hoff, HD)],
                                  bufs.at[slot, t], sem_g).wait()

    def out_slice(i):
        return out_hbm.at[pl.ds(base + i * C, C), pl.ds(hoff, HD)]

    def sum_and_store(i, slot):
        def jstep(j, _):
            for v in range(NVEC):
                sl = pl.ds(v * 16, 16)
                acc = (bufs[slot, 0, j, sl] + bufs[slot, 1, j, sl]
                       + bufs[slot, 2, j, sl] + bufs[slot, 3, j, sl]
                       + bufs[slot, 4, j, sl] + bufs[slot, 5, j, sl])
                bufs[slot, 0, j, sl] = acc
            return 0
        lax.fori_loop(0, C, jstep, 0)
        pltpu.async_copy(bufs.at[slot, 0], out_slice(i), sem_o)

    def wait_out(i, slot):
        pltpu.make_async_copy(bufs.at[slot, 0], out_slice(i), sem_o).wait()

    fire(0, 0)

    def step(i, _):
        slot = lax.rem(i, 2)
        nslot = lax.rem(i + 1, 2)

        @pl.when(i + 1 < NCHUNK)
        def _():
            @pl.when(i >= 1)
            def _():
                wait_out(i - 1, nslot)
            fire(i + 1, nslot)

        wait_gathers(i, slot)
        sum_and_store(i, slot)
        return 0

    lax.fori_loop(0, NCHUNK, step, 0)
    wait_out(NCHUNK - 2, lax.rem(NCHUNK - 2, 2))
    wait_out(NCHUNK - 1, lax.rem(NCHUNK - 1, 2))


@jax.jit
def _run(x0s, y0s, x1s, y1s, x0_t, y0_t, x1_t, y1_t, w_t, h_t):
    mesh = plsc.VectorSubcoreMesh(
        core_axis_name="c", subcore_axis_name="s",
        num_cores=NUM_CORES, num_subcores=NUM_SUBCORES)
    f = pl.kernel(
        _sc_body,
        out_type=jax.ShapeDtypeStruct((N, D), jnp.float32),
        mesh=mesh,
        scratch_types=[
            pltpu.VMEM((TPW,), jnp.int32),
            pltpu.VMEM((TPW,), jnp.int32),
            pltpu.VMEM((TPW,), jnp.int32),
            pltpu.VMEM((TPW,), jnp.int32),
            pltpu.VMEM((2, 6, C, HD), jnp.float32),
            pltpu.SemaphoreType.DMA,
            pltpu.SemaphoreType.DMA,
        ],
    )
    return f(x0s, y0s, x1s, y1s, x0_t, y0_t, x1_t, y1_t, w_t, h_t)


def kernel(bbox, x0_embed, y0_embed, x1_embed, y1_embed, w_embed, h_embed):
    flat = bbox.reshape(N, 4)
    x0s = flat[:, 0]
    y0s = flat[:, 1]
    x1s = flat[:, 2]
    y1s = flat[:, 3]
    out = _run(x0s, y0s, x1s, y1s,
               x0_embed, y0_embed, x1_embed, y1_embed, w_embed, h_embed)
    return out.reshape(B, L, D)
